# Initial kernel scaffold; baseline (speedup 1.0000x reference)
import jax
import jax.numpy as jnp
from jax.experimental import pallas as pl


def kernel(x, Wext1, bext1, gext, beext, Wext2, bext2, Wcls1, bcls1, gcls, becls, Wcls2, bcls2, Wt, bt, emb, ln1g, ln1b, Wqkv, Wo, bo, ln2g, ln2b, Wf1, bf1, Wf2, bf2, lnfg, lnfb, Wfin, bfin):
    raise NotImplementedError("write your pallas kernel here")


# speedup vs baseline: 33.8649x; 33.8649x over previous
import functools
import math

import jax
import jax.numpy as jnp
from jax.experimental import pallas as pl
from jax.experimental.pallas import tpu as pltpu

B, C, H, W = 32, 8, 144, 144
DT, NCLS, TW, OG = 256, 6, 5, 2
ITC = C * H * TW
NBN = B * H
HC = 512


def _k1_body(x_ref, wr_ref, h_ref, stats_ref):
    b = pl.program_id(0)
    xb = x_ref[0]
    acc = jnp.zeros((H, HC), jnp.float32)
    for c in range(C):
        acc = acc + jax.lax.dot_general(
            xb[c], wr_ref[c], (((1,), (0,)), ((), ())),
            preferred_element_type=jnp.float32)
    h_ref[...] = acc[None]
    s = jnp.sum(acc, axis=0, keepdims=True)
    sq = jnp.sum(acc * acc, axis=0, keepdims=True)
    part = jnp.concatenate([s, sq], axis=0)

    @pl.when(b == 0)
    def _init():
        stats_ref[...] = jnp.zeros((2, HC), jnp.float32)

    stats_ref[...] = stats_ref[...] + part


def _k1(x, wr):
    return pl.pallas_call(
        _k1_body,
        grid=(B,),
        in_specs=[
            pl.BlockSpec((1, C, H, W), lambda b: (b, 0, 0, 0)),
            pl.BlockSpec((C, W, HC), lambda b: (0, 0, 0)),
        ],
        out_specs=[
            pl.BlockSpec((1, H, HC), lambda b: (b, 0, 0)),
            pl.BlockSpec((2, HC), lambda b: (0, 0)),
        ],
        out_shape=[
            jax.ShapeDtypeStruct((B, H, HC), jnp.float32),
            jax.ShapeDtypeStruct((2, HC), jnp.float32),
        ],
        compiler_params=pltpu.CompilerParams(
            dimension_semantics=("arbitrary",)),
    )(x, wr)


def _k2_body(with_gather, h_ref, stats_ref, gb_ref, bb_ref, w2e_ref, b2e_ref,
             w2c_ref, b2c_ref, *rest):
    if with_gather:
        x_ref, ext_ref, cls_ref, arg_ref, p_ref = rest
    else:
        ext_ref, cls_ref = rest
    mean = stats_ref[0:1, :] * (1.0 / NBN)
    ex2 = stats_ref[1:2, :] * (1.0 / NBN)
    var = ex2 - mean * mean
    sc = gb_ref[...] * jax.lax.rsqrt(var + 1e-5)
    norm = (h_ref[0] - mean) * sc + bb_ref[...]
    he = norm[:, :DT]
    hc = norm[:, DT:]
    ext_logits = jax.lax.dot_general(
        he, w2e_ref[...], (((1,), (1,)), ((), ())),
        preferred_element_type=jnp.float32) + b2e_ref[...]
    ext_ref[...] = jax.nn.softmax(ext_logits, axis=-1)[None]
    cls_logits = jax.lax.dot_general(
        hc, w2c_ref[...], (((1,), (1,)), ((), ())),
        preferred_element_type=jnp.float32) + b2c_ref[...]
    cls_ref[...] = jax.nn.softmax(cls_logits, axis=-1)[None]
    if with_gather:
        mx = jnp.max(cls_logits, axis=-1, keepdims=True)
        iota = jax.lax.broadcasted_iota(jnp.int32, (H, W), 1)
        arg = jnp.min(jnp.where(cls_logits == mx, iota, 2**30), axis=-1)
        arg_ref[...] = arg.reshape(1, 1, H)
        xb = x_ref[0]
        argc = arg[:, None]
        cols = []
        for t in range(TW):
            m = iota == (argc - OG + t)
            cols.append(jnp.sum(jnp.where(m[None], xb, 0.0), axis=2))
        p_ref[...] = jnp.stack(cols, axis=-1)[None]


def _k2(h, stats, gb, bb, w2e, b2e, w2c, b2c, x=None):
    with_gather = x is not None
    in_specs = [
        pl.BlockSpec((1, H, HC), lambda b: (b, 0, 0)),
        pl.BlockSpec((2, HC), lambda b: (0, 0)),
        pl.BlockSpec((1, HC), lambda b: (0, 0)),
        pl.BlockSpec((1, HC), lambda b: (0, 0)),
        pl.BlockSpec((2, DT), lambda b: (0, 0)),
        pl.BlockSpec((1, 2), lambda b: (0, 0)),
        pl.BlockSpec((H, DT), lambda b: (0, 0)),
        pl.BlockSpec((1, H), lambda b: (0, 0)),
    ]
    out_specs = [
        pl.BlockSpec((1, H, 2), lambda b: (b, 0, 0)),
        pl.BlockSpec((1, H, W), lambda b: (b, 0, 0)),
    ]
    out_shape = [
        jax.ShapeDtypeStruct((B, H, 2), jnp.float32),
        jax.ShapeDtypeStruct((B, H, W), jnp.float32),
    ]
    args = [h, stats, gb, bb, w2e, b2e, w2c, b2c]
    if with_gather:
        in_specs.append(pl.BlockSpec((1, C, H, W), lambda b: (b, 0, 0, 0)))
        out_specs.append(pl.BlockSpec((1, 1, H), lambda b: (b, 0, 0)))
        out_specs.append(pl.BlockSpec((1, C, H, TW), lambda b: (b, 0, 0, 0)))
        out_shape.append(jax.ShapeDtypeStruct((B, 1, H), jnp.int32))
        out_shape.append(jax.ShapeDtypeStruct((B, C, H, TW), jnp.float32))
        args.append(x)
    return pl.pallas_call(
        functools.partial(_k2_body, with_gather),
        grid=(B,),
        in_specs=in_specs,
        out_specs=out_specs,
        out_shape=out_shape,
        compiler_params=pltpu.CompilerParams(
            dimension_semantics=("arbitrary",)),
    )(*args)


def _ln(z, g, b):
    m = jnp.mean(z, axis=-1, keepdims=True)
    d = z - m
    v = jnp.mean(d * d, axis=-1, keepdims=True)
    return d * jax.lax.rsqrt(v + 1e-5) * g + b


def _dot_t(a, w):
    return jax.lax.dot_general(a, w, (((1,), (1,)), ((), ())),
                               preferred_element_type=jnp.float32)


def _k3_body(fp_ref, wt_ref, bt_ref, emb_ref, l1g_ref, l1b_ref, wqkv_ref,
             wo_ref, bo_ref, l2g_ref, l2b_ref, wf1_ref, bf1_ref, wf2_ref,
             bf2_ref, lfg_ref, lfb_ref, wfin_ref, bfin_ref, out_ref):
    R = NCLS * B
    tok = _dot_t(fp_ref[...], wt_ref[...]) + bt_ref[...]
    tokens = (emb_ref[...][:, None, :] + tok[None, :, :]).reshape(R, DT)
    xn = _ln(tokens, l1g_ref[...], l1b_ref[...])
    qkv = _dot_t(xn, wqkv_ref[...])
    ri = jax.lax.broadcasted_iota(jnp.int32, (R, R), 0) % B
    ci = jax.lax.broadcasted_iota(jnp.int32, (R, R), 1) % B
    neg = jnp.where(ri == ci, 0.0, -1e30)
    outs = []
    for hd in range(2):
        q = qkv[:, 32 * hd:32 * hd + 32]
        k = qkv[:, 64 + 32 * hd:96 + 32 * hd]
        v = qkv[:, 128 + 32 * hd:160 + 32 * hd]
        d = _dot_t(q, k) * (32 ** -0.5) + neg
        a = jax.nn.softmax(d, axis=-1)
        outs.append(jax.lax.dot_general(
            a, v, (((1,), (0,)), ((), ())),
            preferred_element_type=jnp.float32))
    o = jnp.concatenate(outs, axis=1)
    t2 = tokens + _dot_t(o, wo_ref[...]) + bo_ref[...]
    ff = _dot_t(_ln(t2, l2g_ref[...], l2b_ref[...]), wf1_ref[...]) + bf1_ref[...]
    ff = ff * 0.5 * (1.0 + jax.lax.erf(ff * (2 ** -0.5)))
    t3 = t2 + _dot_t(ff, wf2_ref[...]) + bf2_ref[...]
    t35 = t3[(NCLS - 1) * B:, :]
    y = _ln(t35, lfg_ref[...], lfb_ref[...])
    out_ref[...] = _dot_t(y, wfin_ref[...]) + bfin_ref[...]


def _k3(fp, wt, bt, emb, l1g, l1b, wqkv, wo, bo, l2g, l2b, wf1, bf1, wf2,
        bf2, lfg, lfb, wfin, bfin):
    args = (fp, wt, bt, emb, l1g, l1b, wqkv, wo, bo, l2g, l2b, wf1, bf1,
            wf2, bf2, lfg, lfb, wfin, bfin)
    return pl.pallas_call(
        _k3_body,
        out_shape=jax.ShapeDtypeStruct((B, ITC), jnp.float32),
    )(*args)


def _k4_body(x_ref, arg_ref, r_ref, out_ref):
    acc = x_ref[0]
    argc = arg_ref[0, 0, :][:, None]
    iota = jax.lax.broadcasted_iota(jnp.int32, (H, W), 1)
    for t in range(TW):
        m = iota == (argc - OG + t)
        rt = r_ref[0, :, :, t][:, :, None]
        acc = jnp.where(m[None], rt, acc)
    out_ref[...] = acc[None]


def _k4(x, arg, r):
    return pl.pallas_call(
        _k4_body,
        grid=(B,),
        in_specs=[
            pl.BlockSpec((1, C, H, W), lambda b: (b, 0, 0, 0)),
            pl.BlockSpec((1, 1, H), lambda b: (b, 0, 0)),
            pl.BlockSpec((1, C, H, TW), lambda b: (b, 0, 0, 0)),
        ],
        out_specs=pl.BlockSpec((1, C, H, W), lambda b: (b, 0, 0, 0)),
        out_shape=jax.ShapeDtypeStruct((B, C, H, W), jnp.float32),
        compiler_params=pltpu.CompilerParams(
            dimension_semantics=("arbitrary",)),
    )(x, arg, r)


def kernel(x, Wext1, bext1, gext, beext, Wext2, bext2, Wcls1, bcls1, gcls,
           becls, Wcls2, bcls2, Wt, bt, emb, ln1g, ln1b, Wqkv, Wo, bo, ln2g,
           ln2b, Wf1, bf1, Wf2, bf2, lnfg, lnfb, Wfin, bfin):
    wr = jnp.transpose(
        jnp.concatenate([Wext1, Wcls1], axis=0).reshape(HC, C, W), (1, 2, 0))
    gb = jnp.concatenate([gext, gcls]).reshape(1, HC)
    bb = jnp.concatenate([beext, becls]).reshape(1, HC)
    b2e = bext2.reshape(1, 2)
    b2c = bcls2.reshape(1, H)

    h1, stats1 = _k1(x, wr)
    ext, cls, arg, p = _k2(h1, stats1, gb, bb, Wext2, b2e, Wcls2, b2c, x=x)
    cond = jnp.mean(ext[:, :, 0]) > 0.3

    def _refine(_):
        fp = p.reshape(B, ITC)
        refined = _k3(fp, Wt, bt.reshape(1, DT), emb, ln1g.reshape(1, DT),
                      ln1b.reshape(1, DT), Wqkv, Wo, bo.reshape(1, DT),
                      ln2g.reshape(1, DT), ln2b.reshape(1, DT), Wf1,
                      bf1.reshape(1, 512), Wf2, bf2.reshape(1, DT),
                      lnfg.reshape(1, DT), lnfb.reshape(1, DT), Wfin,
                      bfin.reshape(1, ITC))
        x2 = _k4(x, arg, refined.reshape(B, C, H, TW))
        h2, stats2 = _k1(x2, wr)
        ext2, cls2 = _k2(h2, stats2, gb, bb, Wext2, b2e, Wcls2, b2c)
        return ext2, cls2

    ext2, cls2 = jax.lax.cond(cond, _refine, lambda _: (ext, cls), None)
    return ext, cls, ext2, cls2
